# baseline (speedup 1.0000x reference)
import jax
import jax.numpy as jnp
from jax import lax
from jax.experimental import pallas as pl
from jax.experimental.pallas import tpu as pltpu
from jax.experimental.pallas import tpu_sc as plsc

VOCAB = 100000
D = 128
NV = D // 16
B = 1024
L = 512
N = B * L

NC = 2
NS = 16
NW = NC * NS

ROWS_PER_W = N // NW
CHUNK = 64
NBUF = 4
NCHUNK = ROWS_PER_W // CHUNK
NOUTER = NCHUNK // NBUF
PCYC = L // CHUNK


_DNUMS = lax.GatherDimensionNumbers(
    offset_dims=(), collapsed_slice_dims=(0,), start_index_map=(0,))


def _lane_sum(v, perms):
  for p in perms:
    v = v + lax.gather(v, p[:, None], dimension_numbers=_DNUMS,
                       slice_sizes=(1,),
                       mode=lax.GatherScatterMode.PROMISE_IN_BOUNDS)
  return v


def _rsqrt_newton(x):
  i = lax.bitcast_convert_type(x, jnp.int32)
  i = jnp.int32(0x5F3759DF) - lax.shift_right_arithmetic(i, 1)
  y = lax.bitcast_convert_type(i, jnp.float32)
  half_x = x * 0.5
  for _ in range(3):
    y = y * (1.5 - half_x * y * y)
  return y


def _fused_body(x_hbm, seg_hbm, table_hbm, segtab_hbm, gamma_hbm, beta_hbm,
                pe_hbm, out_hbm,
                pe_v, segtab_v, gamma_v, beta_v,
                idx_v, segc_v, rows_v,
                isem, ssem, gsem, osem):
  wid = lax.axis_index("s") * NC + lax.axis_index("c")
  base_w = wid * ROWS_PER_W

  pltpu.sync_copy(pe_hbm, pe_v)
  pltpu.sync_copy(segtab_hbm, segtab_v)
  pltpu.sync_copy(gamma_hbm, gamma_v)
  pltpu.sync_copy(beta_hbm, beta_v)

  def idx_copy(b, c):
    src = x_hbm.at[pl.ds(base_w + c * CHUNK, CHUNK)]
    return pltpu.make_async_copy(src, idx_v.at[b], isem.at[b])

  def segc_copy(b, c):
    src = seg_hbm.at[pl.ds(base_w + c * CHUNK, CHUNK)]
    return pltpu.make_async_copy(src, segc_v.at[b], ssem.at[b])

  def gather_copy(b):
    return pltpu.make_async_copy(table_hbm.at[idx_v.at[b]], rows_v.at[b],
                                 gsem.at[b])

  def out_copy(b, c):
    dst = out_hbm.at[pl.ds(base_w + c * CHUNK, CHUNK)]
    return pltpu.make_async_copy(rows_v.at[b], dst, osem.at[b])

  for b in range(NBUF):
    idx_copy(b, b).start()
    segc_copy(b, b).start()

  r0 = [segtab_v[pl.ds(0 * D + 16 * d, 16)] for d in range(NV)]
  d10 = [segtab_v[pl.ds(1 * D + 16 * d, 16)] - r0[d] for d in range(NV)]
  d21 = [segtab_v[pl.ds(2 * D + 16 * d, 16)] -
         segtab_v[pl.ds(1 * D + 16 * d, 16)] for d in range(NV)]
  lanes = lax.broadcasted_iota(jnp.int32, (16,), 0)
  perms = [lanes ^ k for k in (8, 4, 2, 1)]

  def compute_chunk(b, c):
    pbase = (c % PCYC) * CHUNK

    def grp_body(g, _):
      segvec = segc_v[b, pl.ds(g * 16, 16)]
      for j in range(16):
        i = g * 16 + j
        sf = lax.broadcast(segvec[j].astype(jnp.float32), (16,))
        sf1 = jnp.maximum(sf - 1.0, 0.0)
        sf = jnp.minimum(sf, 1.0)
        p = pbase + i
        h = []
        for d in range(NV):
          tok_d = rows_v[b, i, pl.ds(16 * d, 16)]
          pe_d = pe_v[p, pl.ds(16 * d, 16)]
          seg_d = r0[d] + sf * d10[d] + sf1 * d21[d]
          h.append(tok_d + pe_d + seg_d)
        tot = h[0]
        sq = h[0] * h[0]
        for d in range(1, NV):
          tot = tot + h[d]
          sq = sq + h[d] * h[d]
        mean = _lane_sum(tot, perms) * (1.0 / D)
        e2 = _lane_sum(sq, perms) * (1.0 / D)
        var = e2 - mean * mean
        inv = _rsqrt_newton(var + 1e-5)
        shift = mean * inv
        for d in range(NV):
          g_d = gamma_v[pl.ds(16 * d, 16)]
          b_d = beta_v[pl.ds(16 * d, 16)]
          rows_v[b, i, pl.ds(16 * d, 16)] = (h[d] * inv - shift) * g_d + b_d
      return _

    lax.fori_loop(0, CHUNK // 16, grp_body, None)

  def outer(k, _):
    for b in range(NBUF):
      c = k * NBUF + b

      @pl.when(k > 0)
      def _wait_out():
        out_copy(b, c).wait()

      idx_copy(b, c).wait()
      gather_copy(b).start()

    for b in range(NBUF):
      c = k * NBUF + b
      gather_copy(b).wait()
      segc_copy(b, c).wait()
      compute_chunk(b, c)
      out_copy(b, c).start()

      @pl.when(k < NOUTER - 1)
      def _prefetch():
        idx_copy(b, c + NBUF).start()
        segc_copy(b, c + NBUF).start()

    return _

  lax.fori_loop(0, NOUTER, outer, None)

  for b in range(NBUF):
    out_copy(b, (NOUTER - 1) * NBUF + b).wait()


@jax.jit
def _fused(xf, segf, table, segtab, gamma, beta, pe2d):
  mesh = plsc.VectorSubcoreMesh(core_axis_name="c", subcore_axis_name="s")
  return pl.kernel(
      _fused_body,
      out_type=jax.ShapeDtypeStruct((N, D), jnp.float32),
      mesh=mesh,
      scratch_types=[
          pltpu.VMEM((L, D), jnp.float32),
          pltpu.VMEM((3 * D,), jnp.float32),
          pltpu.VMEM((D,), jnp.float32),
          pltpu.VMEM((D,), jnp.float32),
          pltpu.VMEM((NBUF, CHUNK), jnp.int32),
          pltpu.VMEM((NBUF, CHUNK), jnp.int32),
          pltpu.VMEM((NBUF, CHUNK, D), jnp.float32),
          pltpu.SemaphoreType.DMA((NBUF,)),
          pltpu.SemaphoreType.DMA((NBUF,)),
          pltpu.SemaphoreType.DMA((NBUF,)),
          pltpu.SemaphoreType.DMA((NBUF,)),
      ],
  )(xf, segf, table, segtab.reshape(-1), gamma, beta, pe2d)


def kernel(x, seg, tok_table, seg_table, gamma, beta, pe):
  xf = x.reshape(-1)
  segf = seg.reshape(-1)
  pe2d = pe.reshape(pe.shape[1], D)[:L]
  out = _fused(xf, segf, tok_table, seg_table, gamma, beta, pe2d)
  return out.reshape(B, L, D)


# speedup vs baseline: 8.7525x; 8.7525x over previous
import jax
import jax.numpy as jnp
from jax import lax
from jax.experimental import pallas as pl
from jax.experimental.pallas import tpu as pltpu
from jax.experimental.pallas import tpu_sc as plsc

VOCAB = 100000
D = 128
B = 1024
L = 512
N = B * L

NSLICE = 1
M = N // NSLICE

NC = 2
NS = 16
NW = NC * NS

ROWS_PER_W = M // NW
CHUNK = 128
NBUF = 4
NCHUNK = ROWS_PER_W // CHUNK
NOUTER = NCHUNK // NBUF

RB = 2048
SEQ_PER_RB = RB // L


def _sc_gather_body(x_hbm, table_hbm, out_hbm, idx_v, rows_v,
                    isem, gsem, osem):
  wid = lax.axis_index("s") * NC + lax.axis_index("c")
  base_w = wid * ROWS_PER_W

  def idx_copy(b, c):
    src = x_hbm.at[pl.ds(base_w + c * CHUNK, CHUNK)]
    return pltpu.make_async_copy(src, idx_v.at[b], isem.at[b])

  def gather_copy(b):
    return pltpu.make_async_copy(table_hbm.at[idx_v.at[b]], rows_v.at[b],
                                 gsem.at[b])

  def out_copy(b, c):
    dst = out_hbm.at[pl.ds(base_w + c * CHUNK, CHUNK)]
    return pltpu.make_async_copy(rows_v.at[b], dst, osem.at[b])

  for b in range(NBUF):
    idx_copy(b, b).start()

  def outer(k, _):
    for b in range(NBUF):
      c = k * NBUF + b

      @pl.when(k > 0)
      def _wait_prev_out():
        out_copy(b, c - NBUF).wait()

      idx_copy(b, c).wait()
      gather_copy(b).start()

    for b in range(NBUF):
      c = k * NBUF + b
      gather_copy(b).wait()
      out_copy(b, c).start()

      @pl.when(k < NOUTER - 1)
      def _prefetch_idx():
        idx_copy(b, c + NBUF).start()

    return _

  lax.fori_loop(0, NOUTER, outer, None)

  for b in range(NBUF):
    out_copy(b, (NOUTER - 1) * NBUF + b).wait()


@jax.jit
def _sc_gather(xf, table):
  mesh = plsc.VectorSubcoreMesh(core_axis_name="c", subcore_axis_name="s")
  return pl.kernel(
      _sc_gather_body,
      out_type=jax.ShapeDtypeStruct((M, D), jnp.float32),
      mesh=mesh,
      scratch_types=[
          pltpu.VMEM((NBUF, CHUNK), jnp.int32),
          pltpu.VMEM((NBUF, CHUNK, D), jnp.float32),
          pltpu.SemaphoreType.DMA((NBUF,)),
          pltpu.SemaphoreType.DMA((NBUF,)),
          pltpu.SemaphoreType.DMA((NBUF,)),
      ],
  )(xf, table)


def _ln_body(tok_ref, seg_ref, pe_ref, segtab_ref, gamma_ref, beta_ref, o_ref):
  s = seg_ref[0, 0, :][:, None]
  segtab = segtab_ref[...]
  seg_emb = jnp.where(s == 0, segtab[0][None, :],
                      jnp.where(s == 1, segtab[1][None, :],
                                segtab[2][None, :]))
  h = tok_ref[...] + seg_emb
  h = (h.reshape(SEQ_PER_RB, L, D) + pe_ref[...][None]).reshape(RB, D)
  mean = jnp.mean(h, axis=-1, keepdims=True)
  var = jnp.mean(jnp.square(h - mean), axis=-1, keepdims=True)
  inv = lax.rsqrt(var + 1e-5)
  o_ref[...] = (h - mean) * inv * gamma_ref[...] + beta_ref[...]


@jax.jit
def _tc_ln(tok_rows, seg3d, pe2d, seg_table, gamma, beta):
  return pl.pallas_call(
      _ln_body,
      grid=(M // RB,),
      in_specs=[
          pl.BlockSpec((RB, D), lambda i: (i, 0)),
          pl.BlockSpec((1, 1, RB), lambda i: (i, 0, 0)),
          pl.BlockSpec((L, D), lambda i: (0, 0)),
          pl.BlockSpec((3, D), lambda i: (0, 0)),
          pl.BlockSpec((1, D), lambda i: (0, 0)),
          pl.BlockSpec((1, D), lambda i: (0, 0)),
      ],
      out_specs=pl.BlockSpec((RB, D), lambda i: (i, 0)),
      out_shape=jax.ShapeDtypeStruct((M, D), jnp.float32),
  )(tok_rows, seg3d, pe2d, seg_table, gamma, beta)


def kernel(x, seg, tok_table, seg_table, gamma, beta, pe):
  xf = x.reshape(-1)
  seg3d = seg.reshape(-1, 1, RB)
  pe2d = pe.reshape(pe.shape[1], D)[:L]
  g2 = gamma.reshape(1, D)
  b2 = beta.reshape(1, D)
  nblk = M // RB
  outs = []
  for j in range(NSLICE):
    rows_j = _sc_gather(lax.dynamic_slice_in_dim(xf, j * M, M), tok_table)
    outs.append(_tc_ln(rows_j, lax.dynamic_slice_in_dim(seg3d, j * nblk, nblk),
                       pe2d, seg_table, g2, b2))
  out = outs[0] if NSLICE == 1 else jnp.concatenate(outs, axis=0)
  return out.reshape(B, L, D)
